# baseline (speedup 1.0000x reference)
import jax
import jax.numpy as jnp
from jax import lax
from jax.experimental import pallas as pl

DEPTH = 1000
ROWS_PER_BLOCK = 128


def _onehot_block(inp_ref, out_ref):
    idx = inp_ref[...]
    iota = lax.broadcasted_iota(jnp.int32, (idx.shape[0], idx.shape[1], DEPTH), 2)
    out_ref[...] = (idx[:, :, None] == iota).astype(jnp.float32)


def kernel(inputs):
    n, m = inputs.shape
    r = ROWS_PER_BLOCK
    grid = (n // r,)
    return pl.pallas_call(
        _onehot_block,
        grid=grid,
        in_specs=[pl.BlockSpec((r, m), lambda i: (i, 0))],
        out_specs=pl.BlockSpec((r, m, DEPTH), lambda i: (i, 0, 0)),
        out_shape=jax.ShapeDtypeStruct((n, m, DEPTH), jnp.float32),
    )(inputs.astype(jnp.int32))


# speedup vs baseline: 1.0004x; 1.0004x over previous
import jax
import jax.numpy as jnp
from jax import lax
from jax.experimental import pallas as pl
from jax.experimental.pallas import tpu as pltpu

DEPTH = 1000
ROWS_PER_BLOCK = 128


def _onehot_block(inp_ref, out_ref):
    idx = inp_ref[...]
    iota = lax.broadcasted_iota(jnp.int32, (idx.shape[0], idx.shape[1], DEPTH), 2)
    out_ref[...] = (idx[:, :, None] == iota).astype(jnp.float32)


def kernel(inputs):
    n, m = inputs.shape
    r = ROWS_PER_BLOCK
    grid = (n // r,)
    return pl.pallas_call(
        _onehot_block,
        grid=grid,
        in_specs=[pl.BlockSpec((r, m), lambda i: (i, 0))],
        out_specs=pl.BlockSpec((r, m, DEPTH), lambda i: (i, 0, 0)),
        out_shape=jax.ShapeDtypeStruct((n, m, DEPTH), jnp.float32),
        compiler_params=pltpu.CompilerParams(
            dimension_semantics=("parallel",),
        ),
    )(inputs.astype(jnp.int32))
